# Initial kernel scaffold; baseline (speedup 1.0000x reference)
import jax
import jax.numpy as jnp
from jax.experimental import pallas as pl


def kernel(x, Wq, Wk, Wv, Wg, Wo, bo):
    raise NotImplementedError("write your pallas kernel here")


# speedup vs baseline: 5.2741x; 5.2741x over previous
import functools

import jax
import jax.numpy as jnp
from jax.experimental import pallas as pl
from jax.experimental.pallas import tpu as pltpu

_C = 128
_NEWTON = 6


def _dpfp1(z):
    xp = jnp.concatenate([jax.nn.relu(z), jax.nn.relu(-z)], axis=-1)
    rolled = jnp.concatenate([xp[:, -1:], xp[:, :-1]], axis=-1)
    y = xp * rolled
    return y / (jnp.sum(y, axis=-1, keepdims=True) + 1e-6)


def _fwa_body(H, DK, x_ref, wq_ref, wk_ref, wv_ref, wg_ref, wo_ref, bo_ref,
              o_ref, wstate):
    C = _C
    f32 = jnp.float32
    c = pl.program_id(1)

    @pl.when(c == 0)
    def _():
        wstate[...] = jnp.zeros_like(wstate)

    xb = x_ref[...]
    q_all = jnp.dot(xb, wq_ref[...], preferred_element_type=f32)
    k_all = jnp.dot(xb, wk_ref[...], preferred_element_type=f32)
    v_all = jnp.dot(xb, wv_ref[...], preferred_element_type=f32)
    beta_all = jax.nn.sigmoid(
        jnp.dot(xb, wg_ref[...], preferred_element_type=f32))

    ri = jax.lax.broadcasted_iota(jnp.int32, (C, C), 0)
    ci = jax.lax.broadcasted_iota(jnp.int32, (C, C), 1)
    eye = (ri == ci).astype(f32)
    strict = (ri > ci).astype(f32)
    incl = (ri >= ci).astype(f32)

    outs = []
    for h in range(H):
        q = _dpfp1(q_all[:, h * DK:(h + 1) * DK])
        k = _dpfp1(k_all[:, h * DK:(h + 1) * DK])
        v = v_all[:, h * DK:(h + 1) * DK]
        beta = beta_all[:, h:h + 1]
        kb = k * beta
        wt = wstate[h]

        a = strict * jax.lax.dot_general(
            kb, k, (((1,), (1,)), ((), ())), preferred_element_type=f32)
        t = eye - a
        l = eye + a
        for _ in range(_NEWTON):
            t = jnp.dot(t, 2.0 * eye - jnp.dot(l, t, preferred_element_type=f32),
                        preferred_element_type=f32)

        b_rhs = beta * v - jnp.dot(kb, wt, preferred_element_type=f32)
        u = jnp.dot(t, b_rhs, preferred_element_type=f32)
        qk = incl * jax.lax.dot_general(
            q, k, (((1,), (1,)), ((), ())), preferred_element_type=f32)
        o_h = (jnp.dot(q, wt, preferred_element_type=f32)
               + jnp.dot(qk, u, preferred_element_type=f32))
        wstate[h] = wt + jax.lax.dot_general(
            k, u, (((0,), (0,)), ((), ())), preferred_element_type=f32)
        outs.append(o_h)

    o_full = jnp.concatenate(outs, axis=-1)
    o_ref[...] = (jnp.dot(o_full, wo_ref[...], preferred_element_type=f32)
                  + bo_ref[0, :])


def kernel(x, Wq, Wk, Wv, Wg, Wo, bo):
    S, B, D = x.shape
    H = Wg.shape[1]
    DK = Wq.shape[1] // H
    C = _C
    NC = S // C

    xt = jnp.transpose(x, (1, 0, 2))
    body = functools.partial(_fwa_body, H, DK)
    out = pl.pallas_call(
        body,
        grid=(B, NC),
        in_specs=[
            pl.BlockSpec((None, C, D), lambda b, c: (b, c, 0)),
            pl.BlockSpec((D, H * DK), lambda b, c: (0, 0)),
            pl.BlockSpec((D, H * DK), lambda b, c: (0, 0)),
            pl.BlockSpec((D, H * DK), lambda b, c: (0, 0)),
            pl.BlockSpec((D, H), lambda b, c: (0, 0)),
            pl.BlockSpec((D, D), lambda b, c: (0, 0)),
            pl.BlockSpec((1, D), lambda b, c: (0, 0)),
        ],
        out_specs=pl.BlockSpec((None, C, D), lambda b, c: (b, c, 0)),
        out_shape=jax.ShapeDtypeStruct((B, S, D), x.dtype),
        scratch_shapes=[pltpu.VMEM((H, 2 * DK, DK), jnp.float32)],
        compiler_params=pltpu.CompilerParams(
            dimension_semantics=("parallel", "arbitrary")),
    )(xt, Wq, Wk, Wv, Wg, Wo, bo.reshape(1, D))
    return jnp.transpose(out, (1, 0, 2))


# speedup vs baseline: 17.0516x; 3.2331x over previous
import functools

import jax
import jax.numpy as jnp
from jax.experimental import pallas as pl
from jax.experimental.pallas import tpu as pltpu

_C = 128
_NEWTON = 6


def _dpfp1(z):
    xp = jnp.concatenate([jax.nn.relu(z), jax.nn.relu(-z)], axis=-1)
    rolled = jnp.concatenate([xp[:, -1:], xp[:, :-1]], axis=-1)
    y = xp * rolled
    return y / (jnp.sum(y, axis=-1, keepdims=True) + 1e-6)


def _fwa_body(H, DK, x_ref, wq_ref, wk_ref, wv_ref, wg_ref, wo_ref, bo_ref,
              o_ref, wstate):
    C = _C
    f32 = jnp.float32
    c = pl.program_id(1)

    @pl.when(c == 0)
    def _():
        wstate[...] = jnp.zeros_like(wstate)

    xb = x_ref[...]
    q_all = jnp.dot(xb, wq_ref[...], preferred_element_type=f32)
    k_all = jnp.dot(xb, wk_ref[...], preferred_element_type=f32)
    v_all = jnp.dot(xb, wv_ref[...], preferred_element_type=f32)
    beta_all = jax.nn.sigmoid(
        jnp.dot(xb, wg_ref[...], preferred_element_type=f32))

    ri = jax.lax.broadcasted_iota(jnp.int32, (C, C), 0)
    ci = jax.lax.broadcasted_iota(jnp.int32, (C, C), 1)
    eye = (ri == ci).astype(f32)
    strict = (ri > ci).astype(f32)
    incl = (ri >= ci).astype(f32)

    bf16 = jnp.bfloat16
    hs = range(H)
    qs = [_dpfp1(q_all[:, h * DK:(h + 1) * DK]).astype(bf16) for h in hs]
    ks = [_dpfp1(k_all[:, h * DK:(h + 1) * DK]) for h in hs]
    betas = [beta_all[:, h:h + 1] for h in hs]
    khs = [ks[h].astype(bf16) for h in hs]
    kbs = [(ks[h] * betas[h]).astype(bf16) for h in hs]
    wts = [wstate[h] for h in hs]
    wths = [wts[h].astype(bf16) for h in hs]

    a_s = [strict * jax.lax.dot_general(
        kbs[h], khs[h], (((1,), (1,)), ((), ())), preferred_element_type=f32)
        for h in hs]
    ts = [(eye - a_s[h]).astype(bf16) for h in hs]
    ls = [(eye + a_s[h]).astype(bf16) for h in hs]
    for _ in range(_NEWTON):
        inners = [jnp.dot(ls[h], ts[h], preferred_element_type=f32)
                  for h in hs]
        ts = [jnp.dot(ts[h], (2.0 * eye - inners[h]).astype(bf16),
                      preferred_element_type=f32).astype(bf16) for h in hs]

    b_rhss = [(betas[h] * v_all[:, h * DK:(h + 1) * DK]
               - jnp.dot(kbs[h], wths[h], preferred_element_type=f32)
               ).astype(bf16) for h in hs]
    us = [jnp.dot(ts[h], b_rhss[h], preferred_element_type=f32) for h in hs]
    uhs = [us[h].astype(bf16) for h in hs]
    qks = [(incl * jax.lax.dot_general(
        qs[h], khs[h], (((1,), (1,)), ((), ())), preferred_element_type=f32)
            ).astype(bf16) for h in hs]
    o_hs = [jnp.dot(qs[h], wths[h], preferred_element_type=f32)
            + jnp.dot(qks[h], uhs[h], preferred_element_type=f32) for h in hs]
    for h in hs:
        wstate[h] = wts[h] + jax.lax.dot_general(
            khs[h], uhs[h], (((0,), (0,)), ((), ())),
            preferred_element_type=f32)

    o_full = jnp.concatenate(o_hs, axis=-1).astype(bf16)
    o_ref[...] = (jnp.dot(o_full, wo_ref[...].astype(bf16),
                          preferred_element_type=f32)
                  + bo_ref[0, :])


def kernel(x, Wq, Wk, Wv, Wg, Wo, bo):
    S, B, D = x.shape
    H = Wg.shape[1]
    DK = Wq.shape[1] // H
    C = _C
    NC = S // C

    xt = jnp.transpose(x, (1, 0, 2))
    body = functools.partial(_fwa_body, H, DK)
    out = pl.pallas_call(
        body,
        grid=(B, NC),
        in_specs=[
            pl.BlockSpec((None, C, D), lambda b, c: (b, c, 0)),
            pl.BlockSpec((D, H * DK), lambda b, c: (0, 0)),
            pl.BlockSpec((D, H * DK), lambda b, c: (0, 0)),
            pl.BlockSpec((D, H * DK), lambda b, c: (0, 0)),
            pl.BlockSpec((D, H), lambda b, c: (0, 0)),
            pl.BlockSpec((D, D), lambda b, c: (0, 0)),
            pl.BlockSpec((1, D), lambda b, c: (0, 0)),
        ],
        out_specs=pl.BlockSpec((None, C, D), lambda b, c: (b, c, 0)),
        out_shape=jax.ShapeDtypeStruct((B, S, D), x.dtype),
        scratch_shapes=[pltpu.VMEM((H, 2 * DK, DK), jnp.float32)],
        compiler_params=pltpu.CompilerParams(
            dimension_semantics=("parallel", "arbitrary")),
    )(xt, Wq, Wk, Wv, Wg, Wo, bo.reshape(1, D))
    return jnp.transpose(out, (1, 0, 2))


# speedup vs baseline: 17.5216x; 1.0276x over previous
import functools

import jax
import jax.numpy as jnp
from jax.experimental import pallas as pl
from jax.experimental.pallas import tpu as pltpu

_C = 128
_NEWTON = 6


def _dpfp1(z):
    xp = jnp.concatenate([jax.nn.relu(z), jax.nn.relu(-z)], axis=-1)
    rolled = jnp.concatenate([xp[:, -1:], xp[:, :-1]], axis=-1)
    y = xp * rolled
    return y / (jnp.sum(y, axis=-1, keepdims=True) + 1e-6)


def _fwa_body(H, DK, x_ref, x2_ref, wq_ref, wk_ref, wv_ref, wg_ref, wo_ref,
              bo_ref, o_ref, wstate, tscr, qkscr, qscr, kscr, kbscr, bvscr):
    C = _C
    PHI = 2 * DK
    f32 = jnp.float32
    bf16 = jnp.bfloat16
    hs = range(H)
    c = pl.program_id(1)

    ri = jax.lax.broadcasted_iota(jnp.int32, (C, C), 0)
    ci = jax.lax.broadcasted_iota(jnp.int32, (C, C), 1)
    eye = (ri == ci).astype(f32)
    strict = (ri > ci).astype(f32)
    incl = (ri >= ci).astype(f32)

    def _prep(xb_f32, dst):
        xb = xb_f32.astype(bf16)
        q_all = jnp.dot(xb, wq_ref[...].astype(bf16),
                        preferred_element_type=f32)
        k_all = jnp.dot(xb, wk_ref[...].astype(bf16),
                        preferred_element_type=f32)
        v_all = jnp.dot(xb, wv_ref[...].astype(bf16),
                        preferred_element_type=f32)
        beta_all = jax.nn.sigmoid(
            jnp.dot(xb, wg_ref[...].astype(bf16),
                    preferred_element_type=f32))

        qs = [_dpfp1(q_all[:, h * DK:(h + 1) * DK]).astype(bf16) for h in hs]
        ks = [_dpfp1(k_all[:, h * DK:(h + 1) * DK]) for h in hs]
        betas = [beta_all[:, h:h + 1] for h in hs]
        khs = [ks[h].astype(bf16) for h in hs]
        kbs = [(ks[h] * betas[h]).astype(bf16) for h in hs]
        bvs = [(betas[h] * v_all[:, h * DK:(h + 1) * DK]).astype(bf16)
               for h in hs]

        a_s = [strict * jax.lax.dot_general(
            kbs[h], khs[h], (((1,), (1,)), ((), ())),
            preferred_element_type=f32) for h in hs]
        ts = [(eye - a_s[h]).astype(bf16) for h in hs]
        ls = [(eye + a_s[h]).astype(bf16) for h in hs]
        for _ in range(_NEWTON):
            inners = [jnp.dot(ls[h], ts[h], preferred_element_type=f32)
                      for h in hs]
            ts = [jnp.dot(ts[h], (2.0 * eye - inners[h]).astype(bf16),
                          preferred_element_type=f32).astype(bf16)
                  for h in hs]
        qks = [(incl * jax.lax.dot_general(
            qs[h], khs[h], (((1,), (1,)), ((), ())),
            preferred_element_type=f32)).astype(bf16) for h in hs]

        for h in hs:
            tscr[dst, h] = ts[h]
            qkscr[dst, h] = qks[h]
            qscr[dst, :, h * PHI:(h + 1) * PHI] = qs[h]
            kscr[dst, :, h * PHI:(h + 1) * PHI] = khs[h]
            kbscr[dst, :, h * PHI:(h + 1) * PHI] = kbs[h]
            bvscr[dst, :, h * DK:(h + 1) * DK] = bvs[h]

    par = jax.lax.rem(c, 2)
    nxt = 1 - par

    @pl.when(c == 0)
    def _():
        wstate[...] = jnp.zeros_like(wstate)
        _prep(x_ref[...], 0)

    wts = [wstate[h] for h in hs]
    wths = [wts[h].astype(bf16) for h in hs]
    b_rhss = [(bvscr[par, :, h * DK:(h + 1) * DK]
               - jnp.dot(kbscr[par, :, h * PHI:(h + 1) * PHI], wths[h],
                         preferred_element_type=f32)).astype(bf16)
              for h in hs]
    us = [jnp.dot(tscr[par, h], b_rhss[h], preferred_element_type=f32)
          for h in hs]
    uhs = [us[h].astype(bf16) for h in hs]
    o_hs = [jnp.dot(qscr[par, :, h * PHI:(h + 1) * PHI], wths[h],
                    preferred_element_type=f32)
            + jnp.dot(qkscr[par, h], uhs[h], preferred_element_type=f32)
            for h in hs]
    for h in hs:
        wstate[h] = wts[h] + jax.lax.dot_general(
            kscr[par, :, h * PHI:(h + 1) * PHI], uhs[h],
            (((0,), (0,)), ((), ())), preferred_element_type=f32)

    o_full = jnp.concatenate(o_hs, axis=-1).astype(bf16)
    o_ref[...] = (jnp.dot(o_full, wo_ref[...].astype(bf16),
                          preferred_element_type=f32)
                  + bo_ref[0, :])

    _prep(x2_ref[...], nxt)


def kernel(x, Wq, Wk, Wv, Wg, Wo, bo):
    S, B, D = x.shape
    H = Wg.shape[1]
    DK = Wq.shape[1] // H
    PHI = 2 * DK
    C = _C
    NC = S // C

    xt = jnp.transpose(x, (1, 0, 2))
    body = functools.partial(_fwa_body, H, DK)
    out = pl.pallas_call(
        body,
        grid=(B, NC),
        in_specs=[
            pl.BlockSpec((None, C, D), lambda b, c: (b, c, 0)),
            pl.BlockSpec((None, C, D),
                         lambda b, c: (b, jnp.minimum(c + 1, NC - 1), 0)),
            pl.BlockSpec((D, H * DK), lambda b, c: (0, 0)),
            pl.BlockSpec((D, H * DK), lambda b, c: (0, 0)),
            pl.BlockSpec((D, H * DK), lambda b, c: (0, 0)),
            pl.BlockSpec((D, H), lambda b, c: (0, 0)),
            pl.BlockSpec((D, D), lambda b, c: (0, 0)),
            pl.BlockSpec((1, D), lambda b, c: (0, 0)),
        ],
        out_specs=pl.BlockSpec((None, C, D), lambda b, c: (b, c, 0)),
        out_shape=jax.ShapeDtypeStruct((B, S, D), x.dtype),
        scratch_shapes=[
            pltpu.VMEM((H, PHI, DK), jnp.float32),
            pltpu.VMEM((2, H, C, C), jnp.bfloat16),
            pltpu.VMEM((2, H, C, C), jnp.bfloat16),
            pltpu.VMEM((2, C, H * PHI), jnp.bfloat16),
            pltpu.VMEM((2, C, H * PHI), jnp.bfloat16),
            pltpu.VMEM((2, C, H * PHI), jnp.bfloat16),
            pltpu.VMEM((2, C, H * DK), jnp.bfloat16),
        ],
        compiler_params=pltpu.CompilerParams(
            dimension_semantics=("parallel", "arbitrary")),
    )(xt, xt, Wq, Wk, Wv, Wg, Wo, bo.reshape(1, D))
    return jnp.transpose(out, (1, 0, 2))


# speedup vs baseline: 17.5520x; 1.0017x over previous
import functools

import jax
import jax.numpy as jnp
from jax.experimental import pallas as pl
from jax.experimental.pallas import tpu as pltpu

_C = 128
_NEWTON = 6


def _dpfp1(z):
    xp = jnp.concatenate([jax.nn.relu(z), jax.nn.relu(-z)], axis=-1)
    rolled = jnp.concatenate([xp[:, -1:], xp[:, :-1]], axis=-1)
    y = xp * rolled
    return y / (jnp.sum(y, axis=-1, keepdims=True) + 1e-6)


def _fwa_body(H, DK, x_ref, x2_ref, wq_ref, wk_ref, wv_ref, wg_ref, wo_ref,
              bo_ref, o_ref, wstate, tscr, qkscr, qscr, kscr, kbscr, bvscr):
    C = _C
    PHI = 2 * DK
    f32 = jnp.float32
    bf16 = jnp.bfloat16
    hs = range(H)
    c = pl.program_id(1)

    ri = jax.lax.broadcasted_iota(jnp.int32, (C, C), 0)
    ci = jax.lax.broadcasted_iota(jnp.int32, (C, C), 1)
    eye = (ri == ci).astype(f32)
    strict = (ri > ci).astype(f32)
    incl = (ri >= ci).astype(f32)

    def _prep(xb_f32):
        xb = xb_f32.astype(bf16)
        q_all = jnp.dot(xb, wq_ref[...].astype(bf16),
                        preferred_element_type=f32)
        k_all = jnp.dot(xb, wk_ref[...].astype(bf16),
                        preferred_element_type=f32)
        v_all = jnp.dot(xb, wv_ref[...].astype(bf16),
                        preferred_element_type=f32)
        beta_all = jax.nn.sigmoid(
            jnp.dot(xb, wg_ref[...].astype(bf16),
                    preferred_element_type=f32))

        qs = [_dpfp1(q_all[:, h * DK:(h + 1) * DK]).astype(bf16) for h in hs]
        ks = [_dpfp1(k_all[:, h * DK:(h + 1) * DK]) for h in hs]
        betas = [beta_all[:, h:h + 1] for h in hs]
        khs = [ks[h].astype(bf16) for h in hs]
        kbs = [(ks[h] * betas[h]).astype(bf16) for h in hs]
        bvs = [(betas[h] * v_all[:, h * DK:(h + 1) * DK]).astype(bf16)
               for h in hs]

        a_s = [strict * jax.lax.dot_general(
            kbs[h], khs[h], (((1,), (1,)), ((), ())),
            preferred_element_type=f32) for h in hs]
        ts = [(eye - a_s[h]).astype(bf16) for h in hs]
        ls = [(eye + a_s[h]).astype(bf16) for h in hs]
        for _ in range(_NEWTON):
            inners = [jnp.dot(ls[h], ts[h], preferred_element_type=f32)
                      for h in hs]
            ts = [jnp.dot(ts[h], (2.0 * eye - inners[h]).astype(bf16),
                          preferred_element_type=f32).astype(bf16)
                  for h in hs]
        qks = [(incl * jax.lax.dot_general(
            qs[h], khs[h], (((1,), (1,)), ((), ())),
            preferred_element_type=f32)).astype(bf16) for h in hs]

        for h in hs:
            tscr[h] = ts[h]
            qkscr[h] = qks[h]
            qscr[:, h * PHI:(h + 1) * PHI] = qs[h]
            kscr[:, h * PHI:(h + 1) * PHI] = khs[h]
            kbscr[:, h * PHI:(h + 1) * PHI] = kbs[h]
            bvscr[:, h * DK:(h + 1) * DK] = bvs[h]

    @pl.when(c == 0)
    def _():
        wstate[...] = jnp.zeros_like(wstate)
        _prep(x_ref[...])

    wts = [wstate[h] for h in hs]
    wths = [wts[h].astype(bf16) for h in hs]
    b_rhss = [(bvscr[:, h * DK:(h + 1) * DK]
               - jnp.dot(kbscr[:, h * PHI:(h + 1) * PHI], wths[h],
                         preferred_element_type=f32)).astype(bf16)
              for h in hs]
    us = [jnp.dot(tscr[h], b_rhss[h], preferred_element_type=f32)
          for h in hs]
    uhs = [us[h].astype(bf16) for h in hs]
    o_hs = [jnp.dot(qscr[:, h * PHI:(h + 1) * PHI], wths[h],
                    preferred_element_type=f32)
            + jnp.dot(qkscr[h], uhs[h], preferred_element_type=f32)
            for h in hs]
    for h in hs:
        wstate[h] = wts[h] + jax.lax.dot_general(
            kscr[:, h * PHI:(h + 1) * PHI], uhs[h],
            (((0,), (0,)), ((), ())), preferred_element_type=f32)

    o_full = jnp.concatenate(o_hs, axis=-1).astype(bf16)
    o_ref[...] = (jnp.dot(o_full, wo_ref[...].astype(bf16),
                          preferred_element_type=f32)
                  + bo_ref[0, :])

    _prep(x2_ref[...])


def kernel(x, Wq, Wk, Wv, Wg, Wo, bo):
    S, B, D = x.shape
    H = Wg.shape[1]
    DK = Wq.shape[1] // H
    PHI = 2 * DK
    C = _C
    NC = S // C

    xt = jnp.transpose(x, (1, 0, 2))
    body = functools.partial(_fwa_body, H, DK)
    out = pl.pallas_call(
        body,
        grid=(B, NC),
        in_specs=[
            pl.BlockSpec((None, C, D), lambda b, c: (b, c, 0)),
            pl.BlockSpec((None, C, D),
                         lambda b, c: (b, jnp.minimum(c + 1, NC - 1), 0)),
            pl.BlockSpec((D, H * DK), lambda b, c: (0, 0)),
            pl.BlockSpec((D, H * DK), lambda b, c: (0, 0)),
            pl.BlockSpec((D, H * DK), lambda b, c: (0, 0)),
            pl.BlockSpec((D, H), lambda b, c: (0, 0)),
            pl.BlockSpec((D, D), lambda b, c: (0, 0)),
            pl.BlockSpec((1, D), lambda b, c: (0, 0)),
        ],
        out_specs=pl.BlockSpec((None, C, D), lambda b, c: (b, c, 0)),
        out_shape=jax.ShapeDtypeStruct((B, S, D), x.dtype),
        scratch_shapes=[
            pltpu.VMEM((H, PHI, DK), jnp.float32),
            pltpu.VMEM((H, C, C), jnp.bfloat16),
            pltpu.VMEM((H, C, C), jnp.bfloat16),
            pltpu.VMEM((C, H * PHI), jnp.bfloat16),
            pltpu.VMEM((C, H * PHI), jnp.bfloat16),
            pltpu.VMEM((C, H * PHI), jnp.bfloat16),
            pltpu.VMEM((C, H * DK), jnp.bfloat16),
        ],
        compiler_params=pltpu.CompilerParams(
            dimension_semantics=("parallel", "arbitrary")),
    )(xt, xt, Wq, Wk, Wv, Wg, Wo, bo.reshape(1, D))
    return jnp.transpose(out, (1, 0, 2))
